# Initial kernel scaffold; baseline (speedup 1.0000x reference)
import jax
import jax.numpy as jnp
from jax.experimental import pallas as pl


def kernel(x, W):
    raise NotImplementedError("write your pallas kernel here")


# speedup vs baseline: 1.0940x; 1.0940x over previous
import functools

import jax
import jax.numpy as jnp
from jax import lax
from jax.experimental import pallas as pl
from jax.experimental.pallas import tpu as pltpu
from jax.experimental.pallas import tpu_sc as plsc

N = 8192
K = 8192
D = 256
VQ_W = 0.25

BLK_I = 2048
BLK_J = 1024
GRID_I = N // BLK_I
GRID_J = K // BLK_J

SC_CORES = 2
SC_SUBCORES = 16
SC_WORKERS = SC_CORES * SC_SUBCORES
ROWS_PER_WORKER = N // SC_WORKERS


def _argmin_body(x_ref, w_ref, idx_ref, best_d, best_i):
    j = pl.program_id(1)
    xb = x_ref[...]
    wb = w_ref[...]
    c = jnp.sum(xb * xb, axis=1, keepdims=True)
    b = jnp.sum(wb * wb, axis=1)
    m = lax.dot_general(xb, wb, (((1,), (1,)), ((), ())),
                        preferred_element_type=jnp.float32)
    d = (c + b[None, :]) - 2.0 * m

    lm = jnp.min(d, axis=1, keepdims=True)
    ii = lax.broadcasted_iota(jnp.int32, (BLK_I, BLK_J), 1)
    li = jnp.min(jnp.where(d == lm, ii, BLK_J), axis=1, keepdims=True)
    gi = j * BLK_J + li

    @pl.when(j == 0)
    def _():
        best_d[...] = lm
        best_i[...] = gi

    @pl.when(j > 0)
    def _():
        upd = lm < best_d[...]
        best_d[...] = jnp.where(upd, lm, best_d[...])
        best_i[...] = jnp.where(upd, gi, best_i[...])

    @pl.when(j == GRID_J - 1)
    def _():
        idx_ref[0, 0, :] = best_i[:, 0]


def _argmin_call(x, W):
    return pl.pallas_call(
        _argmin_body,
        grid=(GRID_I, GRID_J),
        in_specs=[
            pl.BlockSpec((BLK_I, D), lambda i, j: (i, 0)),
            pl.BlockSpec((BLK_J, D), lambda i, j: (j, 0)),
        ],
        out_specs=pl.BlockSpec((1, 1, BLK_I), lambda i, j: (i, 0, 0)),
        out_shape=jax.ShapeDtypeStruct((GRID_I, 1, BLK_I), jnp.int32),
        scratch_shapes=[
            pltpu.VMEM((BLK_I, 1), jnp.float32),
            pltpu.VMEM((BLK_I, 1), jnp.int32),
        ],
    )(x, W)


def _gather_body(table_hbm, idx_hbm, out_hbm, idx_v, rows_v, sem):
    wid = lax.axis_index("s") * SC_CORES + lax.axis_index("c")
    base = wid * ROWS_PER_WORKER
    pltpu.sync_copy(idx_hbm.at[pl.ds(base, ROWS_PER_WORKER)], idx_v)
    pltpu.async_copy(table_hbm.at[idx_v], rows_v, sem).wait()
    pltpu.sync_copy(rows_v, out_hbm.at[pl.ds(base, ROWS_PER_WORKER)])


def _gather_call(W, idx):
    return pl.kernel(
        _gather_body,
        mesh=plsc.VectorSubcoreMesh(core_axis_name="c", subcore_axis_name="s"),
        out_type=jax.ShapeDtypeStruct((N, D), jnp.float32),
        scratch_types=[
            pltpu.VMEM((ROWS_PER_WORKER,), jnp.int32),
            pltpu.VMEM((ROWS_PER_WORKER, D), jnp.float32),
            pltpu.SemaphoreType.DMA,
        ],
    )(W, idx)


ST_BLK = 1024
ST_GRID = N // ST_BLK


def _st_loss_body(x_ref, q_ref, st_ref, loss_ref, acc):
    r = pl.program_id(0)
    xb = x_ref[...]
    qb = q_ref[...]
    diff = qb - xb
    st_ref[...] = xb + diff

    @pl.when(r == 0)
    def _():
        acc[0, 0] = 0.0

    acc[0, 0] += jnp.sum(diff * diff)

    @pl.when(r == ST_GRID - 1)
    def _():
        mse = acc[0, 0] / (N * D)
        loss_ref[0, 0] = mse * VQ_W + mse


def _st_loss_call(x, q):
    return pl.pallas_call(
        _st_loss_body,
        grid=(ST_GRID,),
        in_specs=[
            pl.BlockSpec((ST_BLK, D), lambda r: (r, 0)),
            pl.BlockSpec((ST_BLK, D), lambda r: (r, 0)),
        ],
        out_specs=[
            pl.BlockSpec((ST_BLK, D), lambda r: (r, 0)),
            pl.BlockSpec(memory_space=pltpu.SMEM),
        ],
        out_shape=[
            jax.ShapeDtypeStruct((N, D), jnp.float32),
            jax.ShapeDtypeStruct((1, 1), jnp.float32),
        ],
        scratch_shapes=[pltpu.SMEM((1, 1), jnp.float32)],
    )(x, q)


def kernel(x, W):
    idx = _argmin_call(x, W).reshape(N)
    q = _gather_call(W, idx)
    st, loss = _st_loss_call(x, q)
    return idx, st, loss.reshape(())
